# Initial kernel scaffold; baseline (speedup 1.0000x reference)
import jax
import jax.numpy as jnp
from jax.experimental import pallas as pl


def kernel(output, item_id, noise):
    raise NotImplementedError("write your pallas kernel here")


# speedup vs baseline: 1.0569x; 1.0569x over previous
import functools

import jax
import jax.numpy as jnp
from jax import lax
from jax.experimental import pallas as pl
from jax.experimental.pallas import tpu as pltpu
from jax.experimental.pallas import tpu_sc as plsc

_B = 16384
_ROWS = 128
_COLS = 128
_NC = 2
_NS = 16
_NW = _NC * _NS
_RPW = _ROWS // _NW
_L = 16


def _noise_body(ids_hbm, outp_hbm, noise_hbm, out_hbm, idx_v, rows_v, out_v, sem):
    wid = lax.axis_index("s") * _NC + lax.axis_index("c")
    r0 = wid * _RPW
    pltpu.sync_copy(ids_hbm.at[pl.ds(r0, _RPW)], idx_v)
    for j in range(_RPW):
        for k in range(_COLS // _L):
            sl = pl.ds(k * _L, _L)
            idx_v[j, sl] = idx_v[j, sl] - 1
    copies = [
        pltpu.async_copy(noise_hbm.at[idx_v.at[j]], rows_v.at[j], sem)
        for j in range(_RPW)
    ]
    pltpu.sync_copy(outp_hbm.at[pl.ds(r0, _RPW)], out_v)
    for cp in copies:
        cp.wait()
    for j in range(_RPW):
        for k in range(_COLS // _L):
            sl = pl.ds(k * _L, _L)
            out_v[j, sl] = out_v[j, sl] + rows_v[j, sl]
    pltpu.sync_copy(out_v, out_hbm.at[pl.ds(r0, _RPW)])


@jax.jit
def kernel(output, item_id, noise):
    ids2 = item_id.reshape(_ROWS, _COLS)
    outp2 = output.reshape(_ROWS, _COLS)
    noise1 = noise.reshape(-1)
    fn = functools.partial(
        pl.kernel,
        mesh=plsc.VectorSubcoreMesh(core_axis_name="c", subcore_axis_name="s"),
        out_type=jax.ShapeDtypeStruct((_ROWS, _COLS), jnp.float32),
        scratch_types=[
            pltpu.VMEM((_RPW, _COLS), jnp.int32),
            pltpu.VMEM((_RPW, _COLS), jnp.float32),
            pltpu.VMEM((_RPW, _COLS), jnp.float32),
            pltpu.SemaphoreType.DMA,
        ],
    )(_noise_body)
    res = fn(ids2, outp2, noise1)
    return res.reshape(_B, 1)
